# baseline (speedup 1.0000x reference)
import jax
import jax.numpy as jnp
from jax.experimental import pallas as pl

H = 64
W = 64
D = 256
HALF = D // 2


def _pos_broadcast(row_embed, col_embed, batch):
    bpb = 2

    def body(row_ref, col_ref, out_ref):
        col = col_ref[...]
        row = row_ref[...]
        top = jnp.broadcast_to(col[None, :, :], (H, W, HALF))
        bot = jnp.broadcast_to(row[:, None, :], (H, W, HALF))
        pos = jnp.concatenate([top, bot], axis=-1)
        out_ref[...] = jnp.broadcast_to(pos[None], (bpb, H, W, D))

    return pl.pallas_call(
        body,
        grid=(batch // bpb,),
        in_specs=[
            pl.BlockSpec((H, HALF), lambda b: (0, 0)),
            pl.BlockSpec((W, HALF), lambda b: (0, 0)),
        ],
        out_specs=pl.BlockSpec((bpb, H, W, D), lambda b: (b, 0, 0, 0)),
        out_shape=jax.ShapeDtypeStruct((batch, H, W, D), jnp.float32),
    )(row_embed, col_embed)


def kernel(x, row_embed, col_embed):
    out_c_minor = _pos_broadcast(row_embed, col_embed, x.shape[0])
    return jnp.transpose(out_c_minor, (0, 3, 1, 2))


# speedup vs baseline: 1.1066x; 1.1066x over previous
import jax
import jax.numpy as jnp
from jax.experimental import pallas as pl

H = 64
W = 64
D = 256
HALF = D // 2


def _pos_broadcast(row_embed, col_embed, batch):
    def body(row_ref, col_ref, out_ref):
        col = col_ref[...]
        row = row_ref[...]
        top = jnp.broadcast_to(col[None, :, :], (H, W, HALF))
        bot = jnp.broadcast_to(row[:, None, :], (H, W, HALF))
        pos = jnp.concatenate([top, bot], axis=-1)
        out_ref[...] = pos[None]

    return pl.pallas_call(
        body,
        grid=(batch,),
        in_specs=[
            pl.BlockSpec((H, HALF), lambda b: (0, 0)),
            pl.BlockSpec((W, HALF), lambda b: (0, 0)),
        ],
        out_specs=pl.BlockSpec((1, H, W, D), lambda b: (b, 0, 0, 0)),
        out_shape=jax.ShapeDtypeStruct((batch, H, W, D), jnp.float32),
    )(row_embed, col_embed)


def kernel(x, row_embed, col_embed):
    out_c_minor = _pos_broadcast(row_embed, col_embed, x.shape[0])
    return jnp.transpose(out_c_minor, (0, 3, 1, 2))
